# baseline (speedup 1.0000x reference)
import jax
import jax.numpy as jnp
from jax.experimental import pallas as pl
from jax.experimental.pallas import tpu as pltpu

_TOKENS = 4096
_DIM = 1024
_EXPERTS = 16
_CAP = 256
_BLK = 256
_NBLK = _TOKENS // _BLK


def _gate_kernel(x_ref, wg_ref, laux_ref, combine_ref, dispatch_ref,
                 carry_ref, me_ref, ce_ref):
    i = pl.program_id(0)

    @pl.when(i == 0)
    def _init():
        carry_ref[...] = jnp.zeros_like(carry_ref)
        me_ref[...] = jnp.zeros_like(me_ref)
        ce_ref[...] = jnp.zeros_like(ce_ref)

    x = x_ref[...]
    wg = wg_ref[...]
    logits = jax.lax.dot_general(
        x, wg, (((1,), (1,)), ((), ())), preferred_element_type=jnp.float32)

    m = jnp.max(logits, axis=1, keepdims=True)
    ex = jnp.exp(logits - m)
    gates = ex / jnp.sum(ex, axis=1, keepdims=True)

    gmax = jnp.max(gates, axis=1, keepdims=True)
    eidx = jax.lax.broadcasted_iota(jnp.int32, gates.shape, 1)
    cand = jnp.where(gates == gmax, eidx, _EXPERTS)
    amax = jnp.min(cand, axis=1, keepdims=True)
    mask1 = (eidx == amax).astype(jnp.float32)

    me_ref[...] += jnp.sum(gates, axis=0, keepdims=True)
    ce_ref[...] += jnp.sum(mask1, axis=0, keepdims=True)

    r = jax.lax.broadcasted_iota(jnp.int32, (_BLK, _BLK), 0)
    c = jax.lax.broadcasted_iota(jnp.int32, (_BLK, _BLK), 1)
    tril = (r > c).astype(jnp.float32)
    prefix = jax.lax.dot_general(
        tril, mask1, (((1,), (0,)), ((), ())),
        preferred_element_type=jnp.float32)
    loc = prefix + carry_ref[...]
    carry_ref[...] += jnp.sum(mask1, axis=0, keepdims=True)

    pos = jnp.sum(loc * mask1, axis=1, keepdims=True)
    keep = (pos < _CAP).astype(jnp.float32)
    gate_val = jnp.sum(gates * mask1, axis=1, keepdims=True) * keep

    gates1 = mask1 * gate_val
    cidx = jax.lax.broadcasted_iota(jnp.int32, (_BLK, _CAP), 1)
    onehot_c = (cidx == pos.astype(jnp.int32)).astype(jnp.float32)

    combine = gates1[:, :, None] * onehot_c[:, None, :]
    combine_ref[...] = combine
    dispatch_ref[...] = combine != 0.0

    @pl.when(i == _NBLK - 1)
    def _fini():
        me = me_ref[...] / _TOKENS
        ce = ce_ref[...] / _TOKENS
        laux_ref[0, 0] = jnp.sum(me * ce) * _EXPERTS


def kernel(input, wg):
    laux, combine, dispatch = pl.pallas_call(
        _gate_kernel,
        grid=(_NBLK,),
        in_specs=[
            pl.BlockSpec((_BLK, _DIM), lambda i: (i, 0)),
            pl.BlockSpec((_EXPERTS, _DIM), lambda i: (0, 0)),
        ],
        out_specs=[
            pl.BlockSpec(memory_space=pltpu.SMEM),
            pl.BlockSpec((_BLK, _EXPERTS, _CAP), lambda i: (i, 0, 0)),
            pl.BlockSpec((_BLK, _EXPERTS, _CAP), lambda i: (i, 0, 0)),
        ],
        out_shape=[
            jax.ShapeDtypeStruct((1, 1), jnp.float32),
            jax.ShapeDtypeStruct((_TOKENS, _EXPERTS, _CAP), jnp.float32),
            jax.ShapeDtypeStruct((_TOKENS, _EXPERTS, _CAP), jnp.bool_),
        ],
        scratch_shapes=[
            pltpu.VMEM((1, _EXPERTS), jnp.float32),
            pltpu.VMEM((1, _EXPERTS), jnp.float32),
            pltpu.VMEM((1, _EXPERTS), jnp.float32),
        ],
        compiler_params=pltpu.CompilerParams(
            dimension_semantics=("arbitrary",)),
    )(input, wg)
    return laux[0, 0], combine, dispatch
